# Initial kernel scaffold; baseline (speedup 1.0000x reference)
import jax
import jax.numpy as jnp
from jax.experimental import pallas as pl


def kernel(x, in_degree, out_degree, in_table, out_table):
    raise NotImplementedError("write your pallas kernel here")


# speedup vs baseline: 2.2187x; 2.2187x over previous
import functools

import jax
import jax.numpy as jnp
from jax import lax
from jax.experimental import pallas as pl
from jax.experimental.pallas import tpu as pltpu
from jax.experimental.pallas import tpu_sc as plsc

_HIDDEN = 128
_N = 100000
_B = 80
_NBLK = _N // _B
_NC = 2
_NS = 16
_NW = _NC * _NS
_ROUNDS = -(-_NBLK // _NW)
_LANES = 16


def _sc_body(x_hbm, ind_hbm, outd_hbm, int_hbm, outt_hbm, o_hbm,
             idx_in, idx_out, rows_in, rows_out, xbuf,
             sem_in, sem_out, sem_x):
    wid = lax.axis_index("s") * _NC + lax.axis_index("c")

    def round_fn(r, carry):
        blk = r * _NW + wid

        @pl.when(blk < _NBLK)
        def _():
            base = blk * _B
            pltpu.sync_copy(ind_hbm.at[pl.ds(base, _B)], idx_in)
            pltpu.sync_copy(outd_hbm.at[pl.ds(base, _B)], idx_out)
            cp_in = pltpu.async_copy(int_hbm.at[idx_in], rows_in, sem_in)
            cp_out = pltpu.async_copy(outt_hbm.at[idx_out], rows_out, sem_out)
            cp_x = pltpu.async_copy(x_hbm.at[pl.ds(base, _B)], xbuf, sem_x)
            cp_in.wait()
            cp_out.wait()
            cp_x.wait()

            def row_fn(i, c):
                for j in range(_HIDDEN // _LANES):
                    sl = pl.ds(j * _LANES, _LANES)
                    xbuf[i, sl] = xbuf[i, sl] + rows_in[i, sl] + rows_out[i, sl]
                return c

            lax.fori_loop(0, _B, row_fn, 0)
            pltpu.sync_copy(xbuf, o_hbm.at[pl.ds(base, _B)])

        return carry

    lax.fori_loop(0, _ROUNDS, round_fn, 0)


@functools.partial(
    pl.kernel,
    mesh=plsc.VectorSubcoreMesh(core_axis_name="c", subcore_axis_name="s"),
    out_type=jax.ShapeDtypeStruct((_N, _HIDDEN), jnp.float32),
    scratch_types=[
        pltpu.VMEM((_B,), jnp.int32),
        pltpu.VMEM((_B,), jnp.int32),
        pltpu.VMEM((_B, _HIDDEN), jnp.float32),
        pltpu.VMEM((_B, _HIDDEN), jnp.float32),
        pltpu.VMEM((_B, _HIDDEN), jnp.float32),
        pltpu.SemaphoreType.DMA,
        pltpu.SemaphoreType.DMA,
        pltpu.SemaphoreType.DMA,
    ],
)
def _centrality_sc(x, ind, outd, int_t, outt, o, *scratch):
    _sc_body(x, ind, outd, int_t, outt, o, *scratch)


def kernel(x, in_degree, out_degree, in_table, out_table):
    return _centrality_sc(
        x,
        in_degree.astype(jnp.int32),
        out_degree.astype(jnp.int32),
        in_table,
        out_table,
    )


# speedup vs baseline: 3.1894x; 1.4375x over previous
import functools

import jax
import jax.numpy as jnp
from jax import lax
from jax.experimental import pallas as pl
from jax.experimental.pallas import tpu as pltpu
from jax.experimental.pallas import tpu_sc as plsc

_HIDDEN = 128
_N = 100000
_NC = 2
_NS = 16
_NW = _NC * _NS
_RPW = _N // _NW
_B = 64
_NB = 48
_TAIL = 48
_SLAB = 3128
_NSETS = 4
_LANES = 16


def _bound(w):
    return pl.multiple_of(((w * _RPW + 4) >> 3) << 3, 8)


def _add_block(xb, ri, ro, nrows):
    def row_fn(i, c):
        for j in range(_HIDDEN // _LANES):
            sl = pl.ds(j * _LANES, _LANES)
            xb[i, sl] = xb[i, sl] + ri[i, sl] + ro[i, sl]
        return c

    lax.fori_loop(0, nrows, row_fn, 0)


def _sc_body(x_hbm, ind_hbm, outd_hbm, int_hbm, outt_hbm, o_hbm,
             slab_in, slab_out, rows_in, rows_out, xbuf, sem_g, sem_s):
    wid = lax.axis_index("s") * _NC + lax.axis_index("c")
    s0 = _bound(wid)
    cnt = _bound(wid + 1) - s0

    pltpu.sync_copy(ind_hbm.at[pl.ds(s0, _SLAB)], slab_in)
    pltpu.sync_copy(outd_hbm.at[pl.ds(s0, _SLAB)], slab_out)

    def prep(m, s):
        pltpu.async_copy(
            int_hbm.at[slab_in.at[pl.ds(m * _B, _B)]], rows_in[s], sem_g[s])
        pltpu.async_copy(
            outt_hbm.at[slab_out.at[pl.ds(m * _B, _B)]], rows_out[s], sem_g[s])
        pltpu.async_copy(x_hbm.at[pl.ds(s0 + m * _B, _B)], xbuf[s], sem_g[s])

    def wait_gathers(s):
        pltpu.make_async_copy(int_hbm.at[slab_in.at[pl.ds(0, _B)]],
                              rows_in[s], sem_g[s]).wait()
        pltpu.make_async_copy(outt_hbm.at[slab_out.at[pl.ds(0, _B)]],
                              rows_out[s], sem_g[s]).wait()
        pltpu.make_async_copy(x_hbm.at[pl.ds(s0, _B)], xbuf[s], sem_g[s]).wait()

    def start_store(m, s):
        pltpu.async_copy(xbuf[s], o_hbm.at[pl.ds(s0 + m * _B, _B)], sem_s[s])

    def wait_store(s):
        pltpu.make_async_copy(xbuf[s], o_hbm.at[pl.ds(s0, _B)], sem_s[s]).wait()

    prep(0, 0)
    prep(1, 1)

    def outer(t, carry):
        for j in range(_NSETS):
            k = t * _NSETS + j
            s = j
            s2 = (j + 2) % _NSETS

            @pl.when(k + 2 < _NB)
            def _():
                @pl.when(k >= 2)
                def _():
                    wait_store(s2)
                prep(k + 2, s2)

            wait_gathers(s)
            _add_block(xbuf[s], rows_in[s], rows_out[s], _B)
            start_store(k, s)
        return carry

    lax.fori_loop(0, _NB // _NSETS, outer, 0)

    for s in range(_NSETS):
        wait_store(s)

    def tail_block(voff, nrows):
        pltpu.async_copy(
            int_hbm.at[slab_in.at[pl.ds(voff, nrows)]],
            rows_in[0].at[pl.ds(0, nrows)], sem_g[0])
        pltpu.async_copy(
            outt_hbm.at[slab_out.at[pl.ds(voff, nrows)]],
            rows_out[0].at[pl.ds(0, nrows)], sem_g[0])
        pltpu.async_copy(
            x_hbm.at[pl.ds(s0 + voff, nrows)], xbuf[0].at[pl.ds(0, nrows)],
            sem_g[0])
        pltpu.make_async_copy(int_hbm.at[slab_in.at[pl.ds(0, nrows)]],
                              rows_in[0].at[pl.ds(0, nrows)], sem_g[0]).wait()
        pltpu.make_async_copy(outt_hbm.at[slab_out.at[pl.ds(0, nrows)]],
                              rows_out[0].at[pl.ds(0, nrows)], sem_g[0]).wait()
        pltpu.make_async_copy(x_hbm.at[pl.ds(s0, nrows)],
                              xbuf[0].at[pl.ds(0, nrows)], sem_g[0]).wait()
        _add_block(xbuf[0], rows_in[0], rows_out[0], nrows)
        pltpu.sync_copy(xbuf[0].at[pl.ds(0, nrows)],
                        o_hbm.at[pl.ds(s0 + voff, nrows)])

    tail_block(_NB * _B, _TAIL)

    @pl.when(cnt == _SLAB)
    def _():
        tail_block(_NB * _B + _TAIL, 8)


@functools.partial(
    pl.kernel,
    mesh=plsc.VectorSubcoreMesh(core_axis_name="c", subcore_axis_name="s"),
    out_type=jax.ShapeDtypeStruct((_N, _HIDDEN), jnp.float32),
    scratch_types=[
        pltpu.VMEM((_SLAB,), jnp.int32),
        pltpu.VMEM((_SLAB,), jnp.int32),
        [pltpu.VMEM((_B, _HIDDEN), jnp.float32) for _ in range(_NSETS)],
        [pltpu.VMEM((_B, _HIDDEN), jnp.float32) for _ in range(_NSETS)],
        [pltpu.VMEM((_B, _HIDDEN), jnp.float32) for _ in range(_NSETS)],
        [pltpu.SemaphoreType.DMA for _ in range(_NSETS)],
        [pltpu.SemaphoreType.DMA for _ in range(_NSETS)],
    ],
)
def _centrality_sc(x, ind, outd, int_t, outt, o,
                   slab_in, slab_out, rows_in, rows_out, xbuf, sem_g, sem_s):
    _sc_body(x, ind, outd, int_t, outt, o,
             slab_in, slab_out, rows_in, rows_out, xbuf, sem_g, sem_s)


def kernel(x, in_degree, out_degree, in_table, out_table):
    return _centrality_sc(
        x,
        in_degree.astype(jnp.int32),
        out_degree.astype(jnp.int32),
        in_table,
        out_table,
    )
